# baseline (speedup 1.0000x reference)
import functools

import jax
import jax.numpy as jnp
from jax import lax
from jax.experimental import pallas as pl
from jax.experimental.pallas import tpu as pltpu
from jax.experimental.pallas import tpu_sc as plsc

B, H, W, C = 4, 192, 192, 96
N = B * H * W * C
NPB = N // B
OUTPB = NPB * 4
OUT = B * OUTPB

NTILES = 16
NCHUNKS = 36
CHUNK = OUT // NCHUNKS
DUMP_PAD = 1024

WSZ = 3072
VPW = WSZ // 16
SLICE = N // NTILES
NWIN = SLICE // WSZ
STAGE = 640
FLUSH = 512
ZBUF = 16384
ZREP = (CHUNK + DUMP_PAD) // NTILES // ZBUF
ZTAIL = (CHUNK + DUMP_PAD) // NTILES - ZREP * ZBUF


def _sc_body(mask_hbm, upd_hbm, out_hbm, mwin, vwin, stage_i, stage_v,
             flush_i, flush_v, zbuf, accum, in_sems):
    cid = lax.axis_index("c")
    sid = lax.axis_index("s")
    slice_base = sid * SLICE
    boff = (slice_base // NPB) * OUTPB

    iota = lax.iota(jnp.int32, 16)
    cvecs = [iota + (r * 16) % 96 + boff for r in range(6)]

    def _z(i, _):
        zbuf[pl.ds(i * 16, 16)] = jnp.zeros((16,), jnp.float32)
        return 0
    lax.fori_loop(0, ZBUF // 16, _z, 0)

    def fire(w, b):
        off = slice_base + w * WSZ
        pltpu.async_copy(mask_hbm.at[pl.ds(off, WSZ)], mwin.at[b], in_sems.at[0, b])
        pltpu.async_copy(upd_hbm.at[pl.ds(off, WSZ)], vwin.at[b], in_sems.at[1, b])

    def wait(w, b):
        off = slice_base + w * WSZ
        pltpu.make_async_copy(mask_hbm.at[pl.ds(off, WSZ)], mwin.at[b], in_sems.at[0, b]).wait()
        pltpu.make_async_copy(upd_hbm.at[pl.ds(off, WSZ)], vwin.at[b], in_sems.at[1, b]).wait()

    def do_flush(n_pairs):
        for j in range(FLUSH // 16):
            flush_i[pl.ds(j * 16, 16)] = stage_i[pl.ds(j * 16, 16)]
            flush_v[pl.ds(j * 16, 16)] = stage_v[pl.ds(j * 16, 16)]
        pltpu.sync_copy(flush_v, accum.at[flush_i], add=True)

    def shift_left(_):
        for j in range((STAGE - FLUSH) // 16):
            stage_i[pl.ds(j * 16, 16)] = stage_i[pl.ds(FLUSH + j * 16, 16)]
            stage_v[pl.ds(j * 16, 16)] = stage_v[pl.ds(FLUSH + j * 16, 16)]

    def round_body(r, _):
        k = 2 * r + cid
        base = k * CHUNK

        zoff = sid * ((CHUNK + DUMP_PAD) // NTILES)
        for j in range(ZREP):
            pltpu.sync_copy(zbuf, accum.at[pl.ds(zoff + j * ZBUF, ZBUF)])
        pltpu.sync_copy(zbuf.at[pl.ds(0, ZTAIL)],
                        accum.at[pl.ds(zoff + ZREP * ZBUF, ZTAIL)])
        plsc.subcore_barrier()

        fire(0, 0)

        def window_body(w, cur_vec):
            b = w % 2
            wait(w, b)

            @pl.when(w + 1 < NWIN)
            def _():
                fire(w + 1, (w + 1) % 2)

            def group_body(gi, cur_vec):
                cur = cur_vec
                for r6 in range(6):
                    v = gi * 6 + r6
                    m = mwin[b, pl.ds(v * 16, 16)]
                    val = vwin[b, pl.ds(v * 16, 16)]
                    g = (m - m % C) + cvecs[r6]
                    loc = g - base
                    inm = (g >= base) & (g < base + CHUNK)
                    ones = jnp.where(inm, 1, 0)
                    pos = cur + plsc.cumsum(ones) - 1
                    plsc.store_scatter(stage_i, [pos], loc, mask=inm)
                    plsc.store_scatter(stage_v, [pos], val, mask=inm)
                    cur = cur + plsc.all_reduce_population_count(inm)
                sc = jnp.max(cur)

                @pl.when(sc >= FLUSH)
                def _():
                    do_flush(sc)
                    shift_left(None)
                return jnp.where(sc >= FLUSH, cur - FLUSH, cur)

            return lax.fori_loop(0, VPW // 6, group_body, cur_vec)

        cur_vec = lax.fori_loop(0, NWIN, window_body, jnp.zeros((16,), jnp.int32))

        cur_s = jnp.max(cur_vec)
        for _rep in range(2):
            for j in range(STAGE // 16):
                posv = iota + j * 16
                cur_i = stage_i[pl.ds(j * 16, 16)]
                stage_i[pl.ds(j * 16, 16)] = jnp.where(
                    posv >= cur_s, CHUNK + posv, cur_i)
            do_flush(cur_s)
            shift_left(None)
            cur_s = jnp.maximum(cur_s - FLUSH, 0)

        plsc.subcore_barrier()

        fl = CHUNK // NTILES
        pltpu.sync_copy(accum.at[pl.ds(sid * fl, fl)],
                        out_hbm.at[pl.ds(base + sid * fl, fl)])
        plsc.subcore_barrier()
        return 0

    lax.fori_loop(0, NCHUNKS // 2, round_body, 0)


@jax.jit
def kernel(updates, mask):
    mask32 = mask.astype(jnp.int32).reshape(-1)
    upd = updates.reshape(-1)

    mesh = plsc.VectorSubcoreMesh(core_axis_name="c", subcore_axis_name="s")
    f = pl.kernel(
        _sc_body,
        out_type=jax.ShapeDtypeStruct((OUT,), jnp.float32),
        mesh=mesh,
        scratch_types=[
            pltpu.VMEM((2, WSZ), jnp.int32),
            pltpu.VMEM((2, WSZ), jnp.float32),
            pltpu.VMEM((STAGE,), jnp.int32),
            pltpu.VMEM((STAGE,), jnp.float32),
            pltpu.VMEM((FLUSH,), jnp.int32),
            pltpu.VMEM((FLUSH,), jnp.float32),
            pltpu.VMEM((ZBUF,), jnp.float32),
            pltpu.VMEM_SHARED((CHUNK + DUMP_PAD,), jnp.float32),
            pltpu.SemaphoreType.DMA((2, 2)),
        ],
        compiler_params=pltpu.CompilerParams(needs_layout_passes=False),
    )
    out = f(mask32, upd)
    return out.reshape(B, H * 2, W * 2, C)


# speedup vs baseline: 1.0827x; 1.0827x over previous
import functools

import jax
import jax.numpy as jnp
from jax import lax
from jax.experimental import pallas as pl
from jax.experimental.pallas import tpu as pltpu
from jax.experimental.pallas import tpu_sc as plsc

B, H, W, C = 4, 192, 192, 96
N = B * H * W * C
NPB = N // B
OUTPB = NPB * 4
OUT = B * OUTPB

NTILES = 16
NCHUNKS = 36
CHUNK = OUT // NCHUNKS
DUMP_PAD = 1024

WSZ = 3072
VPW = WSZ // 16
SLICE = N // NTILES
NWIN = SLICE // WSZ
STAGE = 640
FLUSH = 512
ZBUF = 16384
ZREP = (CHUNK + DUMP_PAD) // NTILES // ZBUF
ZTAIL = (CHUNK + DUMP_PAD) // NTILES - ZREP * ZBUF


def _sc_body(mask_hbm, upd_hbm, out_hbm, mwin, vwin, stage_i, stage_v,
             flush_i, flush_v, zbuf, accum, in_sems):
    cid = lax.axis_index("c")
    sid = lax.axis_index("s")
    slice_base = sid * SLICE
    boff = (slice_base // NPB) * OUTPB

    iota = lax.iota(jnp.int32, 16)
    cvecs = [iota + (r * 16) % 96 + boff for r in range(6)]

    def _z(i, _):
        zbuf[pl.ds(i * 16, 16)] = jnp.zeros((16,), jnp.float32)
        return 0
    lax.fori_loop(0, ZBUF // 16, _z, 0)

    def fire(w, b):
        off = slice_base + w * WSZ
        pltpu.async_copy(mask_hbm.at[pl.ds(off, WSZ)], mwin.at[b], in_sems.at[0, b])
        pltpu.async_copy(upd_hbm.at[pl.ds(off, WSZ)], vwin.at[b], in_sems.at[1, b])

    def wait(w, b):
        off = slice_base + w * WSZ
        pltpu.make_async_copy(mask_hbm.at[pl.ds(off, WSZ)], mwin.at[b], in_sems.at[0, b]).wait()
        pltpu.make_async_copy(upd_hbm.at[pl.ds(off, WSZ)], vwin.at[b], in_sems.at[1, b]).wait()

    def do_flush(n_pairs):
        for j in range(FLUSH // 16):
            flush_i[pl.ds(j * 16, 16)] = stage_i[pl.ds(j * 16, 16)]
            flush_v[pl.ds(j * 16, 16)] = stage_v[pl.ds(j * 16, 16)]
        pltpu.sync_copy(flush_v, accum.at[flush_i], add=True)

    def shift_left(_):
        for j in range((STAGE - FLUSH) // 16):
            stage_i[pl.ds(j * 16, 16)] = stage_i[pl.ds(FLUSH + j * 16, 16)]
            stage_v[pl.ds(j * 16, 16)] = stage_v[pl.ds(FLUSH + j * 16, 16)]

    def round_body(r, _):
        k = 2 * r + cid
        base = k * CHUNK

        zoff = sid * ((CHUNK + DUMP_PAD) // NTILES)
        for j in range(ZREP):
            pltpu.sync_copy(zbuf, accum.at[pl.ds(zoff + j * ZBUF, ZBUF)])
        pltpu.sync_copy(zbuf.at[pl.ds(0, ZTAIL)],
                        accum.at[pl.ds(zoff + ZREP * ZBUF, ZTAIL)])
        plsc.subcore_barrier()

        fire(0, 0)

        def window_body(w, cur_vec):
            b = w % 2
            wait(w, b)

            @pl.when(w + 1 < NWIN)
            def _():
                fire(w + 1, (w + 1) % 2)

            def group_body(gi, cur):
                for r6 in range(6):
                    v = gi * 6 + r6
                    m = mwin[b, pl.ds(v * 16, 16)]
                    val = vwin[b, pl.ds(v * 16, 16)]
                    g = (m - m % C) + cvecs[r6]
                    loc = g - base
                    inm = (g >= base) & (g < base + CHUNK)
                    plsc.store_compressed(stage_i.at[pl.ds(cur, 16)], loc, mask=inm)
                    plsc.store_compressed(stage_v.at[pl.ds(cur, 16)], val, mask=inm)
                    cur = cur + plsc.all_reduce_population_count(inm)[0]

                @pl.when(cur >= FLUSH)
                def _():
                    do_flush(cur)
                    shift_left(None)
                return jnp.where(cur >= FLUSH, cur - FLUSH, cur)

            return lax.fori_loop(0, VPW // 6, group_body, cur_vec)

        cur_s = lax.fori_loop(0, NWIN, window_body, jnp.int32(0))

        for _rep in range(2):
            for j in range(STAGE // 16):
                posv = iota + j * 16
                cur_i = stage_i[pl.ds(j * 16, 16)]
                stage_i[pl.ds(j * 16, 16)] = jnp.where(
                    posv >= cur_s, CHUNK + posv, cur_i)
            do_flush(cur_s)
            shift_left(None)
            cur_s = jnp.maximum(cur_s - FLUSH, 0)

        plsc.subcore_barrier()

        fl = CHUNK // NTILES
        pltpu.sync_copy(accum.at[pl.ds(sid * fl, fl)],
                        out_hbm.at[pl.ds(base + sid * fl, fl)])
        plsc.subcore_barrier()
        return 0

    lax.fori_loop(0, NCHUNKS // 2, round_body, 0)


@jax.jit
def kernel(updates, mask):
    mask32 = mask.astype(jnp.int32).reshape(-1)
    upd = updates.reshape(-1)

    mesh = plsc.VectorSubcoreMesh(core_axis_name="c", subcore_axis_name="s")
    f = pl.kernel(
        _sc_body,
        out_type=jax.ShapeDtypeStruct((OUT,), jnp.float32),
        mesh=mesh,
        scratch_types=[
            pltpu.VMEM((2, WSZ), jnp.int32),
            pltpu.VMEM((2, WSZ), jnp.float32),
            pltpu.VMEM((STAGE,), jnp.int32),
            pltpu.VMEM((STAGE,), jnp.float32),
            pltpu.VMEM((FLUSH,), jnp.int32),
            pltpu.VMEM((FLUSH,), jnp.float32),
            pltpu.VMEM((ZBUF,), jnp.float32),
            pltpu.VMEM_SHARED((CHUNK + DUMP_PAD,), jnp.float32),
            pltpu.SemaphoreType.DMA((2, 2)),
        ],
        compiler_params=pltpu.CompilerParams(needs_layout_passes=False),
    )
    out = f(mask32, upd)
    return out.reshape(B, H * 2, W * 2, C)


# speedup vs baseline: 2.3582x; 2.1781x over previous
import functools

import jax
import jax.numpy as jnp
from jax import lax
from jax.experimental import pallas as pl
from jax.experimental.pallas import tpu as pltpu
from jax.experimental.pallas import tpu_sc as plsc

B, H, W, C = 4, 192, 192, 96
N = B * H * W * C
NPB = N // B
OUTPB = NPB * 4
OUT = B * OUTPB

NTILES = 16
NCHUNKS = 36
CHUNK = OUT // NCHUNKS
DUMP_PAD = 1024

WSZ = 3072
VPW = WSZ // 16
SLICE = N // NTILES
NWIN = SLICE // WSZ
STAGE = 640
FLUSH = 512
ZBUF = 16384
ZREP = (CHUNK + DUMP_PAD) // NTILES // ZBUF
ZTAIL = (CHUNK + DUMP_PAD) // NTILES - ZREP * ZBUF


def _sc_body(mask_hbm, upd_hbm, out_hbm, mwin, vwin, stage_i, stage_v,
             flush_i, flush_v, zbuf, accum, in_sems):
    cid = lax.axis_index("c")
    sid = lax.axis_index("s")
    slice_base = sid * SLICE
    boff = (slice_base // NPB) * OUTPB

    iota = lax.iota(jnp.int32, 16)
    cvecs = [iota + (r * 16) % 96 + boff for r in range(6)]

    def _z(i, _):
        zbuf[pl.ds(i * 16, 16)] = jnp.zeros((16,), jnp.float32)
        return 0
    lax.fori_loop(0, ZBUF // 16, _z, 0)

    def fire(w, b):
        off = slice_base + w * WSZ
        pltpu.async_copy(mask_hbm.at[pl.ds(off, WSZ)], mwin.at[b], in_sems.at[0, b])
        pltpu.async_copy(upd_hbm.at[pl.ds(off, WSZ)], vwin.at[b], in_sems.at[1, b])

    def wait(w, b):
        off = slice_base + w * WSZ
        pltpu.make_async_copy(mask_hbm.at[pl.ds(off, WSZ)], mwin.at[b], in_sems.at[0, b]).wait()
        pltpu.make_async_copy(upd_hbm.at[pl.ds(off, WSZ)], vwin.at[b], in_sems.at[1, b]).wait()

    def do_flush(n_pairs):
        for j in range(FLUSH // 16):
            flush_i[pl.ds(j * 16, 16)] = stage_i[pl.ds(j * 16, 16)]
            flush_v[pl.ds(j * 16, 16)] = stage_v[pl.ds(j * 16, 16)]
        pltpu.sync_copy(flush_v, accum.at[flush_i], add=True)

    def shift_left(_):
        for j in range((STAGE - FLUSH) // 16):
            stage_i[pl.ds(j * 16, 16)] = stage_i[pl.ds(FLUSH + j * 16, 16)]
            stage_v[pl.ds(j * 16, 16)] = stage_v[pl.ds(FLUSH + j * 16, 16)]

    def round_body(r, _):
        k = 2 * r + cid
        base = k * CHUNK

        zoff = sid * ((CHUNK + DUMP_PAD) // NTILES)
        for j in range(ZREP):
            pltpu.sync_copy(zbuf, accum.at[pl.ds(zoff + j * ZBUF, ZBUF)])
        pltpu.sync_copy(zbuf.at[pl.ds(0, ZTAIL)],
                        accum.at[pl.ds(zoff + ZREP * ZBUF, ZTAIL)])
        plsc.subcore_barrier()

        fire(0, 0)

        def window_body(w, cur_vec):
            b = w % 2
            wait(w, b)

            @pl.when(w + 1 < NWIN)
            def _():
                fire(w + 1, (w + 1) % 2)

            def group_body(gi, cur):
                for r6 in range(6):
                    v = gi * 6 + r6
                    m = mwin[b, pl.ds(v * 16, 16)]
                    val = vwin[b, pl.ds(v * 16, 16)]
                    q = (m.astype(jnp.float32) * (1.0 / C)).astype(jnp.int32)
                    r = m - q * C
                    r = jnp.where(r < 0, r + C, r)
                    r = jnp.where(r >= C, r - C, r)
                    g = (m - r) + cvecs[r6]
                    loc = g - base
                    inm = (g >= base) & (g < base + CHUNK)
                    plsc.store_compressed(stage_i.at[pl.ds(cur, 16)], loc, mask=inm)
                    plsc.store_compressed(stage_v.at[pl.ds(cur, 16)], val, mask=inm)
                    cur = cur + plsc.all_reduce_population_count(inm)[0]

                @pl.when(cur >= FLUSH)
                def _():
                    do_flush(cur)
                    shift_left(None)
                return jnp.where(cur >= FLUSH, cur - FLUSH, cur)

            return lax.fori_loop(0, VPW // 6, group_body, cur_vec)

        cur_s = lax.fori_loop(0, NWIN, window_body, jnp.int32(0))

        for _rep in range(2):
            for j in range(STAGE // 16):
                posv = iota + j * 16
                cur_i = stage_i[pl.ds(j * 16, 16)]
                stage_i[pl.ds(j * 16, 16)] = jnp.where(
                    posv >= cur_s, CHUNK + posv, cur_i)
            do_flush(cur_s)
            shift_left(None)
            cur_s = jnp.maximum(cur_s - FLUSH, 0)

        plsc.subcore_barrier()

        fl = CHUNK // NTILES
        pltpu.sync_copy(accum.at[pl.ds(sid * fl, fl)],
                        out_hbm.at[pl.ds(base + sid * fl, fl)])
        plsc.subcore_barrier()
        return 0

    lax.fori_loop(0, NCHUNKS // 2, round_body, 0)


@jax.jit
def kernel(updates, mask):
    mask32 = mask.astype(jnp.int32).reshape(-1)
    upd = updates.reshape(-1)

    mesh = plsc.VectorSubcoreMesh(core_axis_name="c", subcore_axis_name="s")
    f = pl.kernel(
        _sc_body,
        out_type=jax.ShapeDtypeStruct((OUT,), jnp.float32),
        mesh=mesh,
        scratch_types=[
            pltpu.VMEM((2, WSZ), jnp.int32),
            pltpu.VMEM((2, WSZ), jnp.float32),
            pltpu.VMEM((STAGE,), jnp.int32),
            pltpu.VMEM((STAGE,), jnp.float32),
            pltpu.VMEM((FLUSH,), jnp.int32),
            pltpu.VMEM((FLUSH,), jnp.float32),
            pltpu.VMEM((ZBUF,), jnp.float32),
            pltpu.VMEM_SHARED((CHUNK + DUMP_PAD,), jnp.float32),
            pltpu.SemaphoreType.DMA((2, 2)),
        ],
        compiler_params=pltpu.CompilerParams(needs_layout_passes=False),
    )
    out = f(mask32, upd)
    return out.reshape(B, H * 2, W * 2, C)


# speedup vs baseline: 7.0200x; 2.9768x over previous
import jax
import jax.numpy as jnp
from jax import lax
from jax.experimental import pallas as pl
from jax.experimental.pallas import tpu as pltpu
from jax.experimental.pallas import tpu_sc as plsc

B, H, W, C = 4, 192, 192, 96
N = B * H * W * C
NPB = N // B
OUTPB = NPB * 4
OUT = B * OUTPB

NW = 32
NTILES = 16
NB = 36
CHUNK = OUT // NB
DUMP_PAD = 1024

SLICE = N // NW
WSZ = 3072
VPW = WSZ // 16
NWIN = SLICE // WSZ

SCAP = 672
FLUSH = 512
AREA = 460800
REGW = NW * AREA

ZBUF = 16384
ZSLICE = (CHUNK + DUMP_PAD) // NTILES
ZREP = ZSLICE // ZBUF
ZTAIL = ZSLICE - ZREP * ZBUF

RANK_BASE = 1


def _decode(m, cvec):
    q = (m.astype(jnp.float32) * (1.0 / C)).astype(jnp.int32)
    r = m - q * C
    r = jnp.where(r < 0, r + C, r)
    r = jnp.where(r >= C, r - C, r)
    g = (m - r) + cvec
    k = (g.astype(jnp.float32) * (1.0 / CHUNK)).astype(jnp.int32)
    loc = g - k * CHUNK
    neg = loc < 0
    k = jnp.where(neg, k - 1, k)
    loc = jnp.where(neg, loc + CHUNK, loc)
    ovr = loc >= CHUNK
    k = jnp.where(ovr, k + 1, k)
    loc = jnp.where(ovr, loc - CHUNK, loc)
    return k, loc


def _p1_body(mask_hbm, upd_hbm, reg_i, reg_v, hist_hbm,
             mwin, vwin, stage_i, stage_v, cur_ref, offs_ref, wr_ref, in_sems):
    cid = lax.axis_index("c")
    sid = lax.axis_index("s")
    wid = sid * 2 + cid
    slice_base = wid * SLICE
    boff = (slice_base // NPB) * OUTPB
    iota = lax.iota(jnp.int32, 16)
    zero16 = jnp.zeros((16,), jnp.int32)
    lane0 = iota == 0
    cvecs = [iota + r * 16 + boff for r in range(6)]

    for j in range(4):
        cur_ref[pl.ds(j * 16, 16)] = zero16

    def fire(w, b):
        off = slice_base + w * WSZ
        pltpu.async_copy(mask_hbm.at[pl.ds(off, WSZ)], mwin.at[b], in_sems.at[0, b])
        pltpu.async_copy(upd_hbm.at[pl.ds(off, WSZ)], vwin.at[b], in_sems.at[1, b])

    def wait(w, b):
        off = slice_base + w * WSZ
        pltpu.make_async_copy(mask_hbm.at[pl.ds(off, WSZ)], mwin.at[b], in_sems.at[0, b]).wait()
        pltpu.make_async_copy(upd_hbm.at[pl.ds(off, WSZ)], vwin.at[b], in_sems.at[1, b]).wait()

    fire(0, 0)

    def winA(w, _):
        b = w % 2
        wait(w, b)

        @pl.when(w + 1 < NWIN)
        def _():
            fire(w + 1, (w + 1) % 2)

        def grp(gi, _):
            for r6 in range(6):
                v = gi * 6 + r6
                m = mwin[b, pl.ds(v * 16, 16)]
                k, _loc = _decode(m, cvecs[r6])
                hv = plsc.load_gather(cur_ref, [k])
                rank, last = plsc.scan_count(k)
                plsc.store_scatter(cur_ref, [k], hv + rank - RANK_BASE + 1,
                                   mask=last)
            return 0
        return lax.fori_loop(0, VPW // 6, grp, 0)

    lax.fori_loop(0, NWIN, winA, 0)

    pltpu.sync_copy(cur_ref, hist_hbm.at[pl.ds(wid * 64, 64)])
    base = wid * AREA
    carry = base
    for j in range(3):
        n = cur_ref[pl.ds(j * 16, 16)]
        ru = jnp.bitwise_and(n + 511, -512)
        cs = plsc.cumsum(ru)
        offs_ref[pl.ds(j * 16, 16)] = carry + cs - ru
        carry = carry + cs[15]
    offs_ref[pl.ds(48, 16)] = zero16
    for j in range(4):
        wr_ref[pl.ds(j * 16, 16)] = zero16
        cur_ref[pl.ds(j * 16, 16)] = zero16

    def flush_one(k, c):
        ow = plsc.load_gather(offs_ref, [k + zero16])[0]
        wr = plsc.load_gather(wr_ref, [k + zero16])[0]
        dst = pl.multiple_of(ow + wr, FLUSH)
        pltpu.sync_copy(stage_i.at[pl.ds(k * SCAP, FLUSH)],
                        reg_i.at[pl.ds(dst, FLUSH)])
        pltpu.sync_copy(stage_v.at[pl.ds(k * SCAP, FLUSH)],
                        reg_v.at[pl.ds(dst, FLUSH)])
        for j in range((SCAP - FLUSH) // 16):
            stage_i[pl.ds(k * SCAP + j * 16, 16)] = stage_i[pl.ds(k * SCAP + FLUSH + j * 16, 16)]
            stage_v[pl.ds(k * SCAP + j * 16, 16)] = stage_v[pl.ds(k * SCAP + FLUSH + j * 16, 16)]
        plsc.store_scatter(wr_ref, [k + zero16], zero16 + (wr + FLUSH), mask=lane0)
        plsc.store_scatter(cur_ref, [k + zero16], zero16 + (c - FLUSH), mask=lane0)

    fire(0, 0)

    def winB(w, _):
        b = w % 2
        wait(w, b)

        @pl.when(w + 1 < NWIN)
        def _():
            fire(w + 1, (w + 1) % 2)

        def grp(gi, _):
            for r6 in range(6):
                v = gi * 6 + r6
                m = mwin[b, pl.ds(v * 16, 16)]
                val = vwin[b, pl.ds(v * 16, 16)]
                k, loc = _decode(m, cvecs[r6])
                cv = plsc.load_gather(cur_ref, [k])
                rank, last = plsc.scan_count(k)
                pos = cv + rank - RANK_BASE
                addr = k * SCAP + pos
                plsc.store_scatter(stage_i, [addr], loc)
                plsc.store_scatter(stage_v, [addr], val)
                plsc.store_scatter(cur_ref, [k], pos + 1, mask=last)
            c0 = cur_ref[pl.ds(0, 16)]
            c1 = cur_ref[pl.ds(16, 16)]
            c2 = cur_ref[pl.ds(32, 16)]
            hot = (c0 >= FLUSH) | (c1 >= FLUSH) | (c2 >= FLUSH)
            nhot = plsc.all_reduce_population_count(hot)[0]

            @pl.when(nhot > 0)
            def _():
                def fl(k, _):
                    c = plsc.load_gather(cur_ref, [k + zero16])[0]

                    @pl.when(c >= FLUSH)
                    def _():
                        flush_one(k, c)
                    return 0
                lax.fori_loop(0, NB, fl, 0)
            return 0
        return lax.fori_loop(0, VPW // 6, grp, 0)

    lax.fori_loop(0, NWIN, winB, 0)

    def ep(k, _):
        c = plsc.load_gather(cur_ref, [k + zero16])[0]
        for _rep in range(2):
            for j in range(SCAP // 16):
                posv = iota + j * 16
                cur_i = stage_i[pl.ds(k * SCAP + j * 16, 16)]
                stage_i[pl.ds(k * SCAP + j * 16, 16)] = jnp.where(
                    posv >= c, CHUNK + posv, cur_i)

            @pl.when(c > 0)
            def _():
                flush_one(k, c)
            c = jnp.maximum(c - FLUSH, 0)
        return 0
    lax.fori_loop(0, NB, ep, 0)


def _p2_body(reg_i, reg_v, hist_hbm, out_hbm,
             wbuf_i, wbuf_v, hrow, offs2, zbuf, accum):
    cid = lax.axis_index("c")
    sid = lax.axis_index("s")
    iota = lax.iota(jnp.int32, 16)
    zero16 = jnp.zeros((16,), jnp.int32)

    def _z(i, _):
        zbuf[pl.ds(i * 16, 16)] = jnp.zeros((16,), jnp.float32)
        return 0
    lax.fori_loop(0, ZBUF // 16, _z, 0)

    for wi in range(2):
        wrk = sid * 2 + wi
        pltpu.sync_copy(hist_hbm.at[pl.ds(wrk * 64, 64)], hrow.at[pl.ds(wi * 64, 64)])
        carry = wrk * AREA
        for j in range(3):
            n = hrow[pl.ds(wi * 64 + j * 16, 16)]
            ru = jnp.bitwise_and(n + 511, -512)
            cs = plsc.cumsum(ru)
            offs2[pl.ds(wi * 64 + j * 16, 16)] = carry + cs - ru
            carry = carry + cs[15]

    def round_body(r, _):
        k = 2 * r + cid

        zoff = sid * ZSLICE
        for j in range(ZREP):
            pltpu.sync_copy(zbuf, accum.at[pl.ds(zoff + j * ZBUF, ZBUF)])
        pltpu.sync_copy(zbuf.at[pl.ds(0, ZTAIL)],
                        accum.at[pl.ds(zoff + ZREP * ZBUF, ZTAIL)])
        plsc.subcore_barrier()

        for wi in range(2):
            off = pl.multiple_of(
                plsc.load_gather(offs2, [wi * 64 + k + zero16])[0], FLUSH)
            n = plsc.load_gather(hrow, [wi * 64 + k + zero16])[0]
            nwin = lax.shift_right_logical(n + (FLUSH - 1), 9)

            def drain(j, _):
                src = off + j * FLUSH
                pltpu.sync_copy(reg_i.at[pl.ds(src, FLUSH)], wbuf_i)
                pltpu.sync_copy(reg_v.at[pl.ds(src, FLUSH)], wbuf_v)
                pltpu.sync_copy(wbuf_v, accum.at[wbuf_i], add=True)
                return 0
            lax.fori_loop(0, nwin, drain, 0)

        plsc.subcore_barrier()
        fl = CHUNK // NTILES
        pltpu.sync_copy(accum.at[pl.ds(sid * fl, fl)],
                        out_hbm.at[pl.ds(k * CHUNK + sid * fl, fl)])
        plsc.subcore_barrier()
        return 0

    lax.fori_loop(0, NB // 2, round_body, 0)


@jax.jit
def kernel(updates, mask):
    mask32 = mask.astype(jnp.int32).reshape(-1)
    upd = updates.reshape(-1)
    mesh = plsc.VectorSubcoreMesh(core_axis_name="c", subcore_axis_name="s")
    cp = pltpu.CompilerParams(needs_layout_passes=False)

    p1 = pl.kernel(
        _p1_body,
        out_type=[
            jax.ShapeDtypeStruct((REGW,), jnp.int32),
            jax.ShapeDtypeStruct((REGW,), jnp.float32),
            jax.ShapeDtypeStruct((NW * 64,), jnp.int32),
        ],
        mesh=mesh,
        scratch_types=[
            pltpu.VMEM((2, WSZ), jnp.int32),
            pltpu.VMEM((2, WSZ), jnp.float32),
            pltpu.VMEM((NB * SCAP,), jnp.int32),
            pltpu.VMEM((NB * SCAP,), jnp.float32),
            pltpu.VMEM((64,), jnp.int32),
            pltpu.VMEM((64,), jnp.int32),
            pltpu.VMEM((64,), jnp.int32),
            pltpu.SemaphoreType.DMA((2, 2)),
        ],
        compiler_params=cp,
    )
    reg_i, reg_v, hist = p1(mask32, upd)

    p2 = pl.kernel(
        _p2_body,
        out_type=jax.ShapeDtypeStruct((OUT,), jnp.float32),
        mesh=mesh,
        scratch_types=[
            pltpu.VMEM((FLUSH,), jnp.int32),
            pltpu.VMEM((FLUSH,), jnp.float32),
            pltpu.VMEM((128,), jnp.int32),
            pltpu.VMEM((128,), jnp.int32),
            pltpu.VMEM((ZBUF,), jnp.float32),
            pltpu.VMEM_SHARED((CHUNK + DUMP_PAD,), jnp.float32),
        ],
        compiler_params=cp,
    )
    out = p2(reg_i, reg_v, hist)
    return out.reshape(B, H * 2, W * 2, C)


# speedup vs baseline: 9.8542x; 1.4037x over previous
import jax
import jax.numpy as jnp
from jax import lax
from jax.experimental import pallas as pl
from jax.experimental.pallas import tpu as pltpu
from jax.experimental.pallas import tpu_sc as plsc

B, H, W, C = 4, 192, 192, 96
N = B * H * W * C
NPB = N // B
OUTPB = NPB * 4
OUT = B * OUTPB

NW = 32
NTILES = 16
NB = 36
CHUNK = OUT // NB
DUMP_PAD = 1024

SLICE = N // NW
WSZ = 3072
VPW = WSZ // 16
NWIN = SLICE // WSZ

SCAP = 672
FLUSH = 512
AREA = 460800
REGW = NW * AREA

ZBUF = 16384
ZSLICE = (CHUNK + DUMP_PAD) // NTILES
ZREP = ZSLICE // ZBUF
ZTAIL = ZSLICE - ZREP * ZBUF

RANK_BASE = 1


def _decode(m, cvec):
    q = (m.astype(jnp.float32) * (1.0 / C)).astype(jnp.int32)
    r = m - q * C
    r = jnp.where(r < 0, r + C, r)
    r = jnp.where(r >= C, r - C, r)
    g = (m - r) + cvec
    k = (g.astype(jnp.float32) * (1.0 / CHUNK)).astype(jnp.int32)
    loc = g - k * CHUNK
    neg = loc < 0
    k = jnp.where(neg, k - 1, k)
    loc = jnp.where(neg, loc + CHUNK, loc)
    ovr = loc >= CHUNK
    k = jnp.where(ovr, k + 1, k)
    loc = jnp.where(ovr, loc - CHUNK, loc)
    return k, loc


def _p1_body(mask_hbm, upd_hbm, reg_i, reg_v, hist_hbm,
             mwin, vwin, stage_i, stage_v, cur_ref, offs_ref, wr_ref, in_sems):
    cid = lax.axis_index("c")
    sid = lax.axis_index("s")
    wid = sid * 2 + cid
    slice_base = wid * SLICE
    boff = (slice_base // NPB) * OUTPB
    iota = lax.iota(jnp.int32, 16)
    zero16 = jnp.zeros((16,), jnp.int32)
    lane0 = iota == 0
    cvecs = [iota + r * 16 + boff for r in range(6)]

    for j in range(4):
        cur_ref[pl.ds(j * 16, 16)] = zero16

    def fire(w, b):
        off = slice_base + w * WSZ
        pltpu.async_copy(mask_hbm.at[pl.ds(off, WSZ)], mwin.at[b], in_sems.at[0, b])
        pltpu.async_copy(upd_hbm.at[pl.ds(off, WSZ)], vwin.at[b], in_sems.at[1, b])

    def wait(w, b):
        off = slice_base + w * WSZ
        pltpu.make_async_copy(mask_hbm.at[pl.ds(off, WSZ)], mwin.at[b], in_sems.at[0, b]).wait()
        pltpu.make_async_copy(upd_hbm.at[pl.ds(off, WSZ)], vwin.at[b], in_sems.at[1, b]).wait()

    fire(0, 0)

    def winA(w, _):
        b = w % 2
        wait(w, b)

        @pl.when(w + 1 < NWIN)
        def _():
            fire(w + 1, (w + 1) % 2)

        def grp(gi, _):
            for r6 in range(6):
                v = gi * 6 + r6
                m = mwin[b, pl.ds(v * 16, 16)]
                k, _loc = _decode(m, cvecs[r6])
                hv = plsc.load_gather(cur_ref, [k])
                rank, last = plsc.scan_count(k)
                plsc.store_scatter(cur_ref, [k], hv + rank - RANK_BASE + 1,
                                   mask=last)
            return 0
        return lax.fori_loop(0, VPW // 6, grp, 0)

    lax.fori_loop(0, NWIN, winA, 0)

    pltpu.sync_copy(cur_ref, hist_hbm.at[pl.ds(wid * 64, 64)])
    base = wid * AREA
    carry = base
    for j in range(3):
        n = cur_ref[pl.ds(j * 16, 16)]
        ru = jnp.bitwise_and(n + 511, -512)
        cs = plsc.cumsum(ru)
        offs_ref[pl.ds(j * 16, 16)] = carry + cs - ru
        carry = carry + cs[15]
    offs_ref[pl.ds(48, 16)] = zero16
    for j in range(4):
        wr_ref[pl.ds(j * 16, 16)] = zero16
        cur_ref[pl.ds(j * 16, 16)] = zero16

    def flush_one(k, c):
        ow = plsc.load_gather(offs_ref, [k + zero16])[0]
        wr = plsc.load_gather(wr_ref, [k + zero16])[0]
        dst = pl.multiple_of(ow + wr, FLUSH)
        pltpu.sync_copy(stage_i.at[pl.ds(k * SCAP, FLUSH)],
                        reg_i.at[pl.ds(dst, FLUSH)])
        pltpu.sync_copy(stage_v.at[pl.ds(k * SCAP, FLUSH)],
                        reg_v.at[pl.ds(dst, FLUSH)])
        for j in range((SCAP - FLUSH) // 16):
            stage_i[pl.ds(k * SCAP + j * 16, 16)] = stage_i[pl.ds(k * SCAP + FLUSH + j * 16, 16)]
            stage_v[pl.ds(k * SCAP + j * 16, 16)] = stage_v[pl.ds(k * SCAP + FLUSH + j * 16, 16)]
        plsc.store_scatter(wr_ref, [k + zero16], zero16 + (wr + FLUSH), mask=lane0)
        plsc.store_scatter(cur_ref, [k + zero16], zero16 + (c - FLUSH), mask=lane0)

    fire(0, 0)

    def winB(w, _):
        b = w % 2
        wait(w, b)

        @pl.when(w + 1 < NWIN)
        def _():
            fire(w + 1, (w + 1) % 2)

        def grp(gi, _):
            for r6 in range(6):
                v = gi * 6 + r6
                m = mwin[b, pl.ds(v * 16, 16)]
                val = vwin[b, pl.ds(v * 16, 16)]
                k, loc = _decode(m, cvecs[r6])
                cv = plsc.load_gather(cur_ref, [k])
                rank, last = plsc.scan_count(k)
                pos = cv + rank - RANK_BASE
                addr = k * SCAP + pos
                plsc.store_scatter(stage_i, [addr], loc)
                plsc.store_scatter(stage_v, [addr], val)
                plsc.store_scatter(cur_ref, [k], pos + 1, mask=last)
            c0 = cur_ref[pl.ds(0, 16)]
            c1 = cur_ref[pl.ds(16, 16)]
            c2 = cur_ref[pl.ds(32, 16)]
            hot = (c0 >= FLUSH) | (c1 >= FLUSH) | (c2 >= FLUSH)
            nhot = plsc.all_reduce_population_count(hot)[0]

            @pl.when(nhot > 0)
            def _():
                def fl(k, _):
                    c = plsc.load_gather(cur_ref, [k + zero16])[0]

                    @pl.when(c >= FLUSH)
                    def _():
                        flush_one(k, c)
                    return 0
                lax.fori_loop(0, NB, fl, 0)
            return 0
        return lax.fori_loop(0, VPW // 6, grp, 0)

    lax.fori_loop(0, NWIN, winB, 0)

    def ep(k, _):
        c = plsc.load_gather(cur_ref, [k + zero16])[0]
        for _rep in range(2):
            for j in range(SCAP // 16):
                posv = iota + j * 16
                cur_i = stage_i[pl.ds(k * SCAP + j * 16, 16)]
                stage_i[pl.ds(k * SCAP + j * 16, 16)] = jnp.where(
                    posv >= c, CHUNK + posv, cur_i)

            @pl.when(c > 0)
            def _():
                flush_one(k, c)
            c = jnp.maximum(c - FLUSH, 0)
        return 0
    lax.fori_loop(0, NB, ep, 0)


def _p2_body(reg_i, reg_v, hist_hbm, out_hbm,
             wb_i0, wb_v0, wb_i1, wb_v1, hrow, offs2, zbuf, accum, dsem):
    bufs = ((wb_i0, wb_v0), (wb_i1, wb_v1))
    cid = lax.axis_index("c")
    sid = lax.axis_index("s")
    iota = lax.iota(jnp.int32, 16)
    zero16 = jnp.zeros((16,), jnp.int32)

    def _z(i, _):
        zbuf[pl.ds(i * 16, 16)] = jnp.zeros((16,), jnp.float32)
        return 0
    lax.fori_loop(0, ZBUF // 16, _z, 0)

    for wi in range(2):
        wrk = sid * 2 + wi
        pltpu.sync_copy(hist_hbm.at[pl.ds(wrk * 64, 64)], hrow.at[pl.ds(wi * 64, 64)])
        carry = wrk * AREA
        for j in range(3):
            n = hrow[pl.ds(wi * 64 + j * 16, 16)]
            ru = jnp.bitwise_and(n + 511, -512)
            cs = plsc.cumsum(ru)
            offs2[pl.ds(wi * 64 + j * 16, 16)] = carry + cs - ru
            carry = carry + cs[15]

    def round_body(r, _):
        k = 2 * r + cid

        zoff = sid * ZSLICE
        for j in range(ZREP):
            pltpu.sync_copy(zbuf, accum.at[pl.ds(zoff + j * ZBUF, ZBUF)])
        pltpu.sync_copy(zbuf.at[pl.ds(0, ZTAIL)],
                        accum.at[pl.ds(zoff + ZREP * ZBUF, ZTAIL)])
        plsc.subcore_barrier()

        for wi in range(2):
            off = pl.multiple_of(
                plsc.load_gather(offs2, [wi * 64 + k + zero16])[0], FLUSH)
            n = plsc.load_gather(hrow, [wi * 64 + k + zero16])[0]
            nwin = lax.shift_right_logical(n + (FLUSH - 1), 9)

            def fire2(j, bb):
                src = off + j * FLUSH
                pltpu.async_copy(reg_i.at[pl.ds(src, FLUSH)], bufs[bb][0], dsem.at[0, bb])
                pltpu.async_copy(reg_v.at[pl.ds(src, FLUSH)], bufs[bb][1], dsem.at[1, bb])

            def wait2(j, bb):
                src = off + j * FLUSH
                pltpu.make_async_copy(reg_i.at[pl.ds(src, FLUSH)], bufs[bb][0], dsem.at[0, bb]).wait()
                pltpu.make_async_copy(reg_v.at[pl.ds(src, FLUSH)], bufs[bb][1], dsem.at[1, bb]).wait()

            @pl.when(nwin > 0)
            def _():
                fire2(0, 0)

                def drain2(jj, _):
                    for bb in range(2):
                        j = 2 * jj + bb

                        @pl.when(j < nwin)
                        def _():
                            wait2(j, bb)

                            @pl.when(j + 1 < nwin)
                            def _():
                                fire2(j + 1, 1 - bb)
                            pltpu.sync_copy(bufs[bb][1], accum.at[bufs[bb][0]], add=True)
                    return 0
                lax.fori_loop(0, lax.shift_right_logical(nwin + 1, 1), drain2, 0)

        plsc.subcore_barrier()
        fl = CHUNK // NTILES
        pltpu.sync_copy(accum.at[pl.ds(sid * fl, fl)],
                        out_hbm.at[pl.ds(k * CHUNK + sid * fl, fl)])
        plsc.subcore_barrier()
        return 0

    lax.fori_loop(0, NB // 2, round_body, 0)


@jax.jit
def kernel(updates, mask):
    mask32 = mask.astype(jnp.int32).reshape(-1)
    upd = updates.reshape(-1)
    mesh = plsc.VectorSubcoreMesh(core_axis_name="c", subcore_axis_name="s")
    cp = pltpu.CompilerParams(needs_layout_passes=False)

    p1 = pl.kernel(
        _p1_body,
        out_type=[
            jax.ShapeDtypeStruct((REGW,), jnp.int32),
            jax.ShapeDtypeStruct((REGW,), jnp.float32),
            jax.ShapeDtypeStruct((NW * 64,), jnp.int32),
        ],
        mesh=mesh,
        scratch_types=[
            pltpu.VMEM((2, WSZ), jnp.int32),
            pltpu.VMEM((2, WSZ), jnp.float32),
            pltpu.VMEM((NB * SCAP,), jnp.int32),
            pltpu.VMEM((NB * SCAP,), jnp.float32),
            pltpu.VMEM((64,), jnp.int32),
            pltpu.VMEM((64,), jnp.int32),
            pltpu.VMEM((64,), jnp.int32),
            pltpu.SemaphoreType.DMA((2, 2)),
        ],
        compiler_params=cp,
    )
    reg_i, reg_v, hist = p1(mask32, upd)

    p2 = pl.kernel(
        _p2_body,
        out_type=jax.ShapeDtypeStruct((OUT,), jnp.float32),
        mesh=mesh,
        scratch_types=[
            pltpu.VMEM((FLUSH,), jnp.int32),
            pltpu.VMEM((FLUSH,), jnp.float32),
            pltpu.VMEM((FLUSH,), jnp.int32),
            pltpu.VMEM((FLUSH,), jnp.float32),
            pltpu.VMEM((128,), jnp.int32),
            pltpu.VMEM((128,), jnp.int32),
            pltpu.VMEM((ZBUF,), jnp.float32),
            pltpu.VMEM_SHARED((CHUNK + DUMP_PAD,), jnp.float32),
            pltpu.SemaphoreType.DMA((2, 2)),
        ],
        compiler_params=cp,
    )
    out = p2(reg_i, reg_v, hist)
    return out.reshape(B, H * 2, W * 2, C)


# speedup vs baseline: 10.0297x; 1.0178x over previous
import jax
import jax.numpy as jnp
from jax import lax
from jax.experimental import pallas as pl
from jax.experimental.pallas import tpu as pltpu
from jax.experimental.pallas import tpu_sc as plsc

B, H, W, C = 4, 192, 192, 96
N = B * H * W * C
NPB = N // B
OUTPB = NPB * 4
OUT = B * OUTPB

NW = 32
NTILES = 16
NB = 36
CHUNK = OUT // NB
DUMP_PAD = 1024

SLICE = N // NW
WSZ = 3072
VPW = WSZ // 16
NWIN = SLICE // WSZ

SCAP = 704
FLUSH = 512
AREA = 460800
REGW = NW * AREA

ZBUF = 16384
ZSLICE = (CHUNK + DUMP_PAD) // NTILES
ZREP = ZSLICE // ZBUF
ZTAIL = ZSLICE - ZREP * ZBUF

RANK_BASE = 1


def _decode(m, cvec):
    q = (m.astype(jnp.float32) * (1.0 / C)).astype(jnp.int32)
    r = m - q * C
    r = jnp.where(r < 0, r + C, r)
    r = jnp.where(r >= C, r - C, r)
    g = (m - r) + cvec
    k = (g.astype(jnp.float32) * (1.0 / CHUNK)).astype(jnp.int32)
    loc = g - k * CHUNK
    neg = loc < 0
    k = jnp.where(neg, k - 1, k)
    loc = jnp.where(neg, loc + CHUNK, loc)
    ovr = loc >= CHUNK
    k = jnp.where(ovr, k + 1, k)
    loc = jnp.where(ovr, loc - CHUNK, loc)
    return k, loc


def _p1_body(mask_hbm, upd_hbm, reg_i, reg_v, hist_hbm,
             mwin, vwin, stage_i, stage_v, cur_ref, offs_ref, wr_ref, in_sems):
    cid = lax.axis_index("c")
    sid = lax.axis_index("s")
    wid = sid * 2 + cid
    slice_base = wid * SLICE
    boff = (slice_base // NPB) * OUTPB
    iota = lax.iota(jnp.int32, 16)
    zero16 = jnp.zeros((16,), jnp.int32)
    lane0 = iota == 0
    cvecs = [iota + r * 16 + boff for r in range(6)]

    for j in range(4):
        cur_ref[pl.ds(j * 16, 16)] = zero16

    def fire(w, b):
        off = slice_base + w * WSZ
        pltpu.async_copy(mask_hbm.at[pl.ds(off, WSZ)], mwin.at[b], in_sems.at[0, b])
        pltpu.async_copy(upd_hbm.at[pl.ds(off, WSZ)], vwin.at[b], in_sems.at[1, b])

    def wait(w, b):
        off = slice_base + w * WSZ
        pltpu.make_async_copy(mask_hbm.at[pl.ds(off, WSZ)], mwin.at[b], in_sems.at[0, b]).wait()
        pltpu.make_async_copy(upd_hbm.at[pl.ds(off, WSZ)], vwin.at[b], in_sems.at[1, b]).wait()

    fire(0, 0)

    def winA(w, _):
        b = w % 2
        wait(w, b)

        @pl.when(w + 1 < NWIN)
        def _():
            fire(w + 1, (w + 1) % 2)

        def grp(gi, _):
            for r6 in range(6):
                v = gi * 6 + r6
                m = mwin[b, pl.ds(v * 16, 16)]
                k, _loc = _decode(m, cvecs[r6])
                hv = plsc.load_gather(cur_ref, [k])
                rank, last = plsc.scan_count(k)
                plsc.store_scatter(cur_ref, [k], hv + rank - RANK_BASE + 1,
                                   mask=last)
            return 0
        return lax.fori_loop(0, VPW // 6, grp, 0)

    lax.fori_loop(0, NWIN, winA, 0)

    pltpu.sync_copy(cur_ref, hist_hbm.at[pl.ds(wid * 64, 64)])
    base = wid * AREA
    carry = base
    for j in range(3):
        n = cur_ref[pl.ds(j * 16, 16)]
        ru = jnp.bitwise_and(n + 511, -512)
        cs = plsc.cumsum(ru)
        offs_ref[pl.ds(j * 16, 16)] = carry + cs - ru
        carry = carry + cs[15]
    offs_ref[pl.ds(48, 16)] = zero16
    for j in range(4):
        wr_ref[pl.ds(j * 16, 16)] = zero16
        cur_ref[pl.ds(j * 16, 16)] = zero16

    def flush_one(k, c):
        ow = plsc.load_gather(offs_ref, [k + zero16])[0]
        wr = plsc.load_gather(wr_ref, [k + zero16])[0]
        dst = pl.multiple_of(ow + wr, FLUSH)
        pltpu.sync_copy(stage_i.at[pl.ds(k * SCAP, FLUSH)],
                        reg_i.at[pl.ds(dst, FLUSH)])
        pltpu.sync_copy(stage_v.at[pl.ds(k * SCAP, FLUSH)],
                        reg_v.at[pl.ds(dst, FLUSH)])
        for j in range((SCAP - FLUSH) // 16):
            stage_i[pl.ds(k * SCAP + j * 16, 16)] = stage_i[pl.ds(k * SCAP + FLUSH + j * 16, 16)]
            stage_v[pl.ds(k * SCAP + j * 16, 16)] = stage_v[pl.ds(k * SCAP + FLUSH + j * 16, 16)]
        plsc.store_scatter(wr_ref, [k + zero16], zero16 + (wr + FLUSH), mask=lane0)
        plsc.store_scatter(cur_ref, [k + zero16], zero16 + (c - FLUSH), mask=lane0)

    fire(0, 0)

    def winB(w, _):
        b = w % 2
        wait(w, b)

        @pl.when(w + 1 < NWIN)
        def _():
            fire(w + 1, (w + 1) % 2)

        def grp(gi, _):
            for r12 in range(12):
                v = gi * 12 + r12
                m = mwin[b, pl.ds(v * 16, 16)]
                val = vwin[b, pl.ds(v * 16, 16)]
                k, loc = _decode(m, cvecs[r12 % 6])
                cv = plsc.load_gather(cur_ref, [k])
                rank, last = plsc.scan_count(k)
                pos = cv + rank - RANK_BASE
                addr = k * SCAP + pos
                plsc.store_scatter(stage_i, [addr], loc)
                plsc.store_scatter(stage_v, [addr], val)
                plsc.store_scatter(cur_ref, [k], pos + 1, mask=last)
            c0 = cur_ref[pl.ds(0, 16)]
            c1 = cur_ref[pl.ds(16, 16)]
            c2 = cur_ref[pl.ds(32, 16)]
            hot = (c0 >= FLUSH) | (c1 >= FLUSH) | (c2 >= FLUSH)
            nhot = plsc.all_reduce_population_count(hot)[0]

            @pl.when(nhot > 0)
            def _():
                def fl(k, _):
                    c = plsc.load_gather(cur_ref, [k + zero16])[0]

                    @pl.when(c >= FLUSH)
                    def _():
                        flush_one(k, c)
                    return 0
                lax.fori_loop(0, NB, fl, 0)
            return 0
        return lax.fori_loop(0, VPW // 12, grp, 0)

    lax.fori_loop(0, NWIN, winB, 0)

    def ep(k, _):
        c = plsc.load_gather(cur_ref, [k + zero16])[0]
        for _rep in range(2):
            for j in range(SCAP // 16):
                posv = iota + j * 16
                cur_i = stage_i[pl.ds(k * SCAP + j * 16, 16)]
                stage_i[pl.ds(k * SCAP + j * 16, 16)] = jnp.where(
                    posv >= c, CHUNK + posv, cur_i)

            @pl.when(c > 0)
            def _():
                flush_one(k, c)
            c = jnp.maximum(c - FLUSH, 0)
        return 0
    lax.fori_loop(0, NB, ep, 0)


def _p2_body(reg_i, reg_v, hist_hbm, out_hbm,
             wb_i0, wb_v0, wb_i1, wb_v1, hrow, offs2, zbuf, accum, dsem):
    bufs = ((wb_i0, wb_v0), (wb_i1, wb_v1))
    cid = lax.axis_index("c")
    sid = lax.axis_index("s")
    iota = lax.iota(jnp.int32, 16)
    zero16 = jnp.zeros((16,), jnp.int32)

    def _z(i, _):
        zbuf[pl.ds(i * 16, 16)] = jnp.zeros((16,), jnp.float32)
        return 0
    lax.fori_loop(0, ZBUF // 16, _z, 0)

    for wi in range(2):
        wrk = sid * 2 + wi
        pltpu.sync_copy(hist_hbm.at[pl.ds(wrk * 64, 64)], hrow.at[pl.ds(wi * 64, 64)])
        carry = wrk * AREA
        for j in range(3):
            n = hrow[pl.ds(wi * 64 + j * 16, 16)]
            ru = jnp.bitwise_and(n + 511, -512)
            cs = plsc.cumsum(ru)
            offs2[pl.ds(wi * 64 + j * 16, 16)] = carry + cs - ru
            carry = carry + cs[15]

    def round_body(r, _):
        k = 2 * r + cid

        zoff = sid * ZSLICE
        for j in range(ZREP):
            pltpu.sync_copy(zbuf, accum.at[pl.ds(zoff + j * ZBUF, ZBUF)])
        pltpu.sync_copy(zbuf.at[pl.ds(0, ZTAIL)],
                        accum.at[pl.ds(zoff + ZREP * ZBUF, ZTAIL)])
        plsc.subcore_barrier()

        for wi in range(2):
            off = pl.multiple_of(
                plsc.load_gather(offs2, [wi * 64 + k + zero16])[0], FLUSH)
            n = plsc.load_gather(hrow, [wi * 64 + k + zero16])[0]
            nwin = lax.shift_right_logical(n + (FLUSH - 1), 9)

            def fire2(j, bb):
                src = off + j * FLUSH
                pltpu.async_copy(reg_i.at[pl.ds(src, FLUSH)], bufs[bb][0], dsem.at[0, bb])
                pltpu.async_copy(reg_v.at[pl.ds(src, FLUSH)], bufs[bb][1], dsem.at[1, bb])

            def wait2(j, bb):
                src = off + j * FLUSH
                pltpu.make_async_copy(reg_i.at[pl.ds(src, FLUSH)], bufs[bb][0], dsem.at[0, bb]).wait()
                pltpu.make_async_copy(reg_v.at[pl.ds(src, FLUSH)], bufs[bb][1], dsem.at[1, bb]).wait()

            @pl.when(nwin > 0)
            def _():
                fire2(0, 0)

                def drain2(jj, _):
                    for bb in range(2):
                        j = 2 * jj + bb

                        @pl.when(j < nwin)
                        def _():
                            wait2(j, bb)

                            @pl.when(j + 1 < nwin)
                            def _():
                                fire2(j + 1, 1 - bb)
                            pltpu.sync_copy(bufs[bb][1], accum.at[bufs[bb][0]], add=True)
                    return 0
                lax.fori_loop(0, lax.shift_right_logical(nwin + 1, 1), drain2, 0)

        plsc.subcore_barrier()
        fl = CHUNK // NTILES
        pltpu.sync_copy(accum.at[pl.ds(sid * fl, fl)],
                        out_hbm.at[pl.ds(k * CHUNK + sid * fl, fl)])
        plsc.subcore_barrier()
        return 0

    lax.fori_loop(0, NB // 2, round_body, 0)


@jax.jit
def kernel(updates, mask):
    mask32 = mask.astype(jnp.int32).reshape(-1)
    upd = updates.reshape(-1)
    mesh = plsc.VectorSubcoreMesh(core_axis_name="c", subcore_axis_name="s")
    cp = pltpu.CompilerParams(needs_layout_passes=False)

    p1 = pl.kernel(
        _p1_body,
        out_type=[
            jax.ShapeDtypeStruct((REGW,), jnp.int32),
            jax.ShapeDtypeStruct((REGW,), jnp.float32),
            jax.ShapeDtypeStruct((NW * 64,), jnp.int32),
        ],
        mesh=mesh,
        scratch_types=[
            pltpu.VMEM((2, WSZ), jnp.int32),
            pltpu.VMEM((2, WSZ), jnp.float32),
            pltpu.VMEM((NB * SCAP,), jnp.int32),
            pltpu.VMEM((NB * SCAP,), jnp.float32),
            pltpu.VMEM((64,), jnp.int32),
            pltpu.VMEM((64,), jnp.int32),
            pltpu.VMEM((64,), jnp.int32),
            pltpu.SemaphoreType.DMA((2, 2)),
        ],
        compiler_params=cp,
    )
    reg_i, reg_v, hist = p1(mask32, upd)

    p2 = pl.kernel(
        _p2_body,
        out_type=jax.ShapeDtypeStruct((OUT,), jnp.float32),
        mesh=mesh,
        scratch_types=[
            pltpu.VMEM((FLUSH,), jnp.int32),
            pltpu.VMEM((FLUSH,), jnp.float32),
            pltpu.VMEM((FLUSH,), jnp.int32),
            pltpu.VMEM((FLUSH,), jnp.float32),
            pltpu.VMEM((128,), jnp.int32),
            pltpu.VMEM((128,), jnp.int32),
            pltpu.VMEM((ZBUF,), jnp.float32),
            pltpu.VMEM_SHARED((CHUNK + DUMP_PAD,), jnp.float32),
            pltpu.SemaphoreType.DMA((2, 2)),
        ],
        compiler_params=cp,
    )
    out = p2(reg_i, reg_v, hist)
    return out.reshape(B, H * 2, W * 2, C)


# speedup vs baseline: 10.0636x; 1.0034x over previous
import jax
import jax.numpy as jnp
from jax import lax
from jax.experimental import pallas as pl
from jax.experimental.pallas import tpu as pltpu
from jax.experimental.pallas import tpu_sc as plsc

B, H, W, C = 4, 192, 192, 96
N = B * H * W * C
NPB = N // B
OUTPB = NPB * 4
OUT = B * OUTPB

NW = 32
NTILES = 16
NB = 36
CHUNK = OUT // NB
DUMP_PAD = 1024

SLICE = N // NW
WSZ = 3072
VPW = WSZ // 16
NWIN = SLICE // WSZ

SCAP = 704
FLUSH = 512
AREA = 460800
REGW = NW * AREA

ZBUF = 16384
ZSLICE = (CHUNK + DUMP_PAD) // NTILES
ZREP = ZSLICE // ZBUF
ZTAIL = ZSLICE - ZREP * ZBUF

RANK_BASE = 1


def _decode(m, cvec):
    q = (m.astype(jnp.float32) * (1.0 / C)).astype(jnp.int32)
    r = m - q * C
    r = jnp.where(r < 0, r + C, r)
    r = jnp.where(r >= C, r - C, r)
    g = (m - r) + cvec
    k = (g.astype(jnp.float32) * (1.0 / CHUNK)).astype(jnp.int32)
    loc = g - k * CHUNK
    neg = loc < 0
    k = jnp.where(neg, k - 1, k)
    loc = jnp.where(neg, loc + CHUNK, loc)
    ovr = loc >= CHUNK
    k = jnp.where(ovr, k + 1, k)
    loc = jnp.where(ovr, loc - CHUNK, loc)
    return k, loc


def _p1_body(mask_hbm, upd_hbm, reg_i, reg_v, hist_hbm,
             mwin, vwin, stage_i, stage_v, cur_ref, offs_ref, wr_ref, in_sems):
    cid = lax.axis_index("c")
    sid = lax.axis_index("s")
    wid = sid * 2 + cid
    slice_base = wid * SLICE
    boff = (slice_base // NPB) * OUTPB
    iota = lax.iota(jnp.int32, 16)
    zero16 = jnp.zeros((16,), jnp.int32)
    lane0 = iota == 0
    cvecs = [iota + r * 16 + boff for r in range(6)]

    for j in range(4):
        cur_ref[pl.ds(j * 16, 16)] = zero16

    def fire(w, b, with_val=True):
        off = slice_base + w * WSZ
        pltpu.async_copy(mask_hbm.at[pl.ds(off, WSZ)], mwin.at[b], in_sems.at[0, b])
        if with_val:
            pltpu.async_copy(upd_hbm.at[pl.ds(off, WSZ)], vwin.at[b], in_sems.at[1, b])

    def wait(w, b, with_val=True):
        off = slice_base + w * WSZ
        pltpu.make_async_copy(mask_hbm.at[pl.ds(off, WSZ)], mwin.at[b], in_sems.at[0, b]).wait()
        if with_val:
            pltpu.make_async_copy(upd_hbm.at[pl.ds(off, WSZ)], vwin.at[b], in_sems.at[1, b]).wait()

    fire(0, 0, with_val=False)

    def winA(w, _):
        b = w % 2
        wait(w, b, with_val=False)

        @pl.when(w + 1 < NWIN)
        def _():
            fire(w + 1, (w + 1) % 2, with_val=False)

        def grp(gi, _):
            for r6 in range(6):
                v = gi * 6 + r6
                m = mwin[b, pl.ds(v * 16, 16)]
                k, _loc = _decode(m, cvecs[r6])
                hv = plsc.load_gather(cur_ref, [k])
                rank, last = plsc.scan_count(k)
                plsc.store_scatter(cur_ref, [k], hv + rank - RANK_BASE + 1,
                                   mask=last)
            return 0
        return lax.fori_loop(0, VPW // 6, grp, 0)

    lax.fori_loop(0, NWIN, winA, 0)

    pltpu.sync_copy(cur_ref, hist_hbm.at[pl.ds(wid * 64, 64)])
    base = wid * AREA
    carry = base
    for j in range(3):
        n = cur_ref[pl.ds(j * 16, 16)]
        ru = jnp.bitwise_and(n + 511, -512)
        cs = plsc.cumsum(ru)
        offs_ref[pl.ds(j * 16, 16)] = carry + cs - ru
        carry = carry + cs[15]
    offs_ref[pl.ds(48, 16)] = zero16
    for j in range(4):
        wr_ref[pl.ds(j * 16, 16)] = zero16
        cur_ref[pl.ds(j * 16, 16)] = zero16

    def flush_one(k, c):
        ow = plsc.load_gather(offs_ref, [k + zero16])[0]
        wr = plsc.load_gather(wr_ref, [k + zero16])[0]
        dst = pl.multiple_of(ow + wr, FLUSH)
        pltpu.sync_copy(stage_i.at[pl.ds(k * SCAP, FLUSH)],
                        reg_i.at[pl.ds(dst, FLUSH)])
        pltpu.sync_copy(stage_v.at[pl.ds(k * SCAP, FLUSH)],
                        reg_v.at[pl.ds(dst, FLUSH)])
        for j in range((SCAP - FLUSH) // 16):
            stage_i[pl.ds(k * SCAP + j * 16, 16)] = stage_i[pl.ds(k * SCAP + FLUSH + j * 16, 16)]
            stage_v[pl.ds(k * SCAP + j * 16, 16)] = stage_v[pl.ds(k * SCAP + FLUSH + j * 16, 16)]
        plsc.store_scatter(wr_ref, [k + zero16], zero16 + (wr + FLUSH), mask=lane0)
        plsc.store_scatter(cur_ref, [k + zero16], zero16 + (c - FLUSH), mask=lane0)

    fire(0, 0)

    def winB(w, _):
        b = w % 2
        wait(w, b)

        @pl.when(w + 1 < NWIN)
        def _():
            fire(w + 1, (w + 1) % 2)

        def grp(gi, _):
            for r12 in range(12):
                v = gi * 12 + r12
                m = mwin[b, pl.ds(v * 16, 16)]
                val = vwin[b, pl.ds(v * 16, 16)]
                k, loc = _decode(m, cvecs[r12 % 6])
                cv = plsc.load_gather(cur_ref, [k])
                rank, last = plsc.scan_count(k)
                pos = cv + rank - RANK_BASE
                addr = k * SCAP + pos
                plsc.store_scatter(stage_i, [addr], loc)
                plsc.store_scatter(stage_v, [addr], val)
                plsc.store_scatter(cur_ref, [k], pos + 1, mask=last)
            c0 = cur_ref[pl.ds(0, 16)]
            c1 = cur_ref[pl.ds(16, 16)]
            c2 = cur_ref[pl.ds(32, 16)]
            hot = (c0 >= FLUSH) | (c1 >= FLUSH) | (c2 >= FLUSH)
            nhot = plsc.all_reduce_population_count(hot)[0]

            @pl.when(nhot > 0)
            def _():
                def fl(k, _):
                    c = plsc.load_gather(cur_ref, [k + zero16])[0]

                    @pl.when(c >= FLUSH)
                    def _():
                        flush_one(k, c)
                    return 0
                lax.fori_loop(0, NB, fl, 0)
            return 0
        return lax.fori_loop(0, VPW // 12, grp, 0)

    lax.fori_loop(0, NWIN, winB, 0)

    def ep(k, _):
        c = plsc.load_gather(cur_ref, [k + zero16])[0]
        for _rep in range(2):
            for j in range(SCAP // 16):
                posv = iota + j * 16
                cur_i = stage_i[pl.ds(k * SCAP + j * 16, 16)]
                stage_i[pl.ds(k * SCAP + j * 16, 16)] = jnp.where(
                    posv >= c, CHUNK + posv, cur_i)

            @pl.when(c > 0)
            def _():
                flush_one(k, c)
            c = jnp.maximum(c - FLUSH, 0)
        return 0
    lax.fori_loop(0, NB, ep, 0)


def _p2_body(reg_i, reg_v, hist_hbm, out_hbm,
             wb_i0, wb_v0, wb_i1, wb_v1, hrow, offs2, zbuf, accum, dsem):
    bufs = ((wb_i0, wb_v0), (wb_i1, wb_v1))
    cid = lax.axis_index("c")
    sid = lax.axis_index("s")
    iota = lax.iota(jnp.int32, 16)
    zero16 = jnp.zeros((16,), jnp.int32)

    def _z(i, _):
        zbuf[pl.ds(i * 16, 16)] = jnp.zeros((16,), jnp.float32)
        return 0
    lax.fori_loop(0, ZBUF // 16, _z, 0)

    for wi in range(2):
        wrk = sid * 2 + wi
        pltpu.sync_copy(hist_hbm.at[pl.ds(wrk * 64, 64)], hrow.at[pl.ds(wi * 64, 64)])
        carry = wrk * AREA
        for j in range(3):
            n = hrow[pl.ds(wi * 64 + j * 16, 16)]
            ru = jnp.bitwise_and(n + 511, -512)
            cs = plsc.cumsum(ru)
            offs2[pl.ds(wi * 64 + j * 16, 16)] = carry + cs - ru
            carry = carry + cs[15]

    def round_body(r, _):
        k = 2 * r + cid

        zoff = sid * ZSLICE
        for j in range(ZREP):
            pltpu.sync_copy(zbuf, accum.at[pl.ds(zoff + j * ZBUF, ZBUF)])
        pltpu.sync_copy(zbuf.at[pl.ds(0, ZTAIL)],
                        accum.at[pl.ds(zoff + ZREP * ZBUF, ZTAIL)])
        plsc.subcore_barrier()

        for wi in range(2):
            off = pl.multiple_of(
                plsc.load_gather(offs2, [wi * 64 + k + zero16])[0], FLUSH)
            n = plsc.load_gather(hrow, [wi * 64 + k + zero16])[0]
            nwin = lax.shift_right_logical(n + (FLUSH - 1), 9)

            def fire2(j, bb):
                src = off + j * FLUSH
                pltpu.async_copy(reg_i.at[pl.ds(src, FLUSH)], bufs[bb][0], dsem.at[0, bb])
                pltpu.async_copy(reg_v.at[pl.ds(src, FLUSH)], bufs[bb][1], dsem.at[1, bb])

            def wait2(j, bb):
                src = off + j * FLUSH
                pltpu.make_async_copy(reg_i.at[pl.ds(src, FLUSH)], bufs[bb][0], dsem.at[0, bb]).wait()
                pltpu.make_async_copy(reg_v.at[pl.ds(src, FLUSH)], bufs[bb][1], dsem.at[1, bb]).wait()

            @pl.when(nwin > 0)
            def _():
                fire2(0, 0)

                def drain2(jj, _):
                    for bb in range(2):
                        j = 2 * jj + bb

                        @pl.when(j < nwin)
                        def _():
                            wait2(j, bb)

                            @pl.when(j + 1 < nwin)
                            def _():
                                fire2(j + 1, 1 - bb)
                            pltpu.sync_copy(bufs[bb][1], accum.at[bufs[bb][0]], add=True)
                    return 0
                lax.fori_loop(0, lax.shift_right_logical(nwin + 1, 1), drain2, 0)

        plsc.subcore_barrier()
        fl = CHUNK // NTILES
        pltpu.sync_copy(accum.at[pl.ds(sid * fl, fl)],
                        out_hbm.at[pl.ds(k * CHUNK + sid * fl, fl)])
        plsc.subcore_barrier()
        return 0

    lax.fori_loop(0, NB // 2, round_body, 0)


@jax.jit
def kernel(updates, mask):
    mask32 = mask.astype(jnp.int32).reshape(-1)
    upd = updates.reshape(-1)
    mesh = plsc.VectorSubcoreMesh(core_axis_name="c", subcore_axis_name="s")
    cp = pltpu.CompilerParams(needs_layout_passes=False)

    p1 = pl.kernel(
        _p1_body,
        out_type=[
            jax.ShapeDtypeStruct((REGW,), jnp.int32),
            jax.ShapeDtypeStruct((REGW,), jnp.float32),
            jax.ShapeDtypeStruct((NW * 64,), jnp.int32),
        ],
        mesh=mesh,
        scratch_types=[
            pltpu.VMEM((2, WSZ), jnp.int32),
            pltpu.VMEM((2, WSZ), jnp.float32),
            pltpu.VMEM((NB * SCAP,), jnp.int32),
            pltpu.VMEM((NB * SCAP,), jnp.float32),
            pltpu.VMEM((64,), jnp.int32),
            pltpu.VMEM((64,), jnp.int32),
            pltpu.VMEM((64,), jnp.int32),
            pltpu.SemaphoreType.DMA((2, 2)),
        ],
        compiler_params=cp,
    )
    reg_i, reg_v, hist = p1(mask32, upd)

    p2 = pl.kernel(
        _p2_body,
        out_type=jax.ShapeDtypeStruct((OUT,), jnp.float32),
        mesh=mesh,
        scratch_types=[
            pltpu.VMEM((FLUSH,), jnp.int32),
            pltpu.VMEM((FLUSH,), jnp.float32),
            pltpu.VMEM((FLUSH,), jnp.int32),
            pltpu.VMEM((FLUSH,), jnp.float32),
            pltpu.VMEM((128,), jnp.int32),
            pltpu.VMEM((128,), jnp.int32),
            pltpu.VMEM((ZBUF,), jnp.float32),
            pltpu.VMEM_SHARED((CHUNK + DUMP_PAD,), jnp.float32),
            pltpu.SemaphoreType.DMA((2, 2)),
        ],
        compiler_params=cp,
    )
    out = p2(reg_i, reg_v, hist)
    return out.reshape(B, H * 2, W * 2, C)
